# baseline (device time: 47420 ns/iter reference)
import jax
import jax.numpy as jnp
from jax import lax
from jax.experimental import pallas as pl
from jax.experimental.pallas import tpu as pltpu


def kernel(dy, W):
    m, _ = dy.shape
    d = W.shape[0]

    def body(dy_ref, w_ref, out_ref, acc_ref, recv_ref,
             semA_send, semA_recv, semB_send, semB_recv):
        my_x = lax.axis_index("x")
        my_y = lax.axis_index("y")
        my_z = lax.axis_index("z")
        pA = my_z ^ 1
        pB = my_z ^ 2

        barrier = pltpu.get_barrier_semaphore()
        for p in (pA, pB):
            pl.semaphore_signal(
                barrier, inc=1,
                device_id=(my_x, my_y, p),
                device_id_type=pl.DeviceIdType.MESH,
            )
        pl.semaphore_wait(barrier, 2)

        acc_ref[...] = lax.dot_general(
            dy_ref[...], w_ref[...],
            dimension_numbers=(((1,), (1,)), ((), ())),
            preferred_element_type=jnp.float32,
        )

        rdma_a = pltpu.make_async_remote_copy(
            src_ref=acc_ref,
            dst_ref=recv_ref.at[0],
            send_sem=semA_send,
            recv_sem=semA_recv,
            device_id=(my_x, my_y, pA),
            device_id_type=pl.DeviceIdType.MESH,
        )
        rdma_a.start()
        rdma_a.wait()
        acc_ref[...] = acc_ref[...] + recv_ref[0]

        rdma_b = pltpu.make_async_remote_copy(
            src_ref=acc_ref,
            dst_ref=recv_ref.at[1],
            send_sem=semB_send,
            recv_sem=semB_recv,
            device_id=(my_x, my_y, pB),
            device_id_type=pl.DeviceIdType.MESH,
        )
        rdma_b.start()
        rdma_b.wait()
        out_ref[...] = acc_ref[...] + recv_ref[1]

    return pl.pallas_call(
        body,
        out_shape=jax.ShapeDtypeStruct((m, d), jnp.float32),
        in_specs=[
            pl.BlockSpec(memory_space=pltpu.VMEM),
            pl.BlockSpec(memory_space=pltpu.VMEM),
        ],
        out_specs=pl.BlockSpec(memory_space=pltpu.VMEM),
        scratch_shapes=[
            pltpu.VMEM((m, d), jnp.float32),
            pltpu.VMEM((2, m, d), jnp.float32),
            pltpu.SemaphoreType.DMA,
            pltpu.SemaphoreType.DMA,
            pltpu.SemaphoreType.DMA,
            pltpu.SemaphoreType.DMA,
        ],
        compiler_params=pltpu.CompilerParams(collective_id=0),
    )(dy, W)


# device time: 40519 ns/iter; 1.1703x vs baseline; 1.1703x over previous
import jax
import jax.numpy as jnp
from jax import lax
from jax.experimental import pallas as pl
from jax.experimental.pallas import tpu as pltpu

N_CHUNK = 8


def kernel(dy, W):
    m, _ = dy.shape
    d = W.shape[0]
    rows = m // N_CHUNK

    def body(dy_ref, w_ref, out_ref, acc_ref, mid_ref, buf_edge, buf_peer,
             sem_edge_send, sem_edge_recv, sem_peer_send, sem_peer_recv,
             sem_final_send, sem_final_recv):
        my_x = lax.axis_index("x")
        my_y = lax.axis_index("y")
        my_z = lax.axis_index("z")
        is_edge = jnp.logical_or(my_z == 0, my_z == 3)

        barrier = pltpu.get_barrier_semaphore()

        @pl.when(my_z < 3)
        def _():
            pl.semaphore_signal(
                barrier, inc=1,
                device_id=(my_x, my_y, my_z + 1),
                device_id_type=pl.DeviceIdType.MESH,
            )

        @pl.when(my_z > 0)
        def _():
            pl.semaphore_signal(
                barrier, inc=1,
                device_id=(my_x, my_y, my_z - 1),
                device_id_type=pl.DeviceIdType.MESH,
            )

        pl.semaphore_wait(barrier, jnp.where(is_edge, 1, 2))

        acc_ref[...] = lax.dot_general(
            dy_ref[...], w_ref[...],
            dimension_numbers=(((1,), (1,)), ((), ())),
            preferred_element_type=jnp.float32,
        )

        @pl.when(is_edge)
        def _():
            inner = jnp.where(my_z == 0, 1, 2)
            sends = []
            for k in range(N_CHUNK):
                sl = pl.ds(k * rows, rows)
                s = pltpu.make_async_remote_copy(
                    src_ref=acc_ref.at[sl],
                    dst_ref=buf_edge.at[k],
                    send_sem=sem_edge_send.at[k],
                    recv_sem=sem_edge_recv.at[k],
                    device_id=(my_x, my_y, inner),
                    device_id_type=pl.DeviceIdType.MESH,
                )
                s.start()
                sends.append(s)
            for k in range(N_CHUNK):
                sl = pl.ds(k * rows, rows)
                r = pltpu.make_async_remote_copy(
                    src_ref=acc_ref.at[sl],
                    dst_ref=out_ref.at[sl],
                    send_sem=sem_final_send.at[k],
                    recv_sem=sem_final_recv.at[k],
                    device_id=(my_x, my_y, inner),
                    device_id_type=pl.DeviceIdType.MESH,
                )
                r.wait_recv()
            for s in sends:
                s.wait_send()

        @pl.when(jnp.logical_not(is_edge))
        def _():
            edge_nbr = jnp.where(my_z == 1, 0, 3)
            peer = jnp.where(my_z == 1, 2, 1)
            exchanges = []
            final_sends = []
            for k in range(N_CHUNK):
                sl = pl.ds(k * rows, rows)
                re = pltpu.make_async_remote_copy(
                    src_ref=acc_ref.at[sl],
                    dst_ref=buf_edge.at[k],
                    send_sem=sem_edge_send.at[k],
                    recv_sem=sem_edge_recv.at[k],
                    device_id=(my_x, my_y, edge_nbr),
                    device_id_type=pl.DeviceIdType.MESH,
                )
                re.wait_recv()
                mid_ref[k] = acc_ref[sl, :] + buf_edge[k]
                ex = pltpu.make_async_remote_copy(
                    src_ref=mid_ref.at[k],
                    dst_ref=buf_peer.at[k],
                    send_sem=sem_peer_send.at[k],
                    recv_sem=sem_peer_recv.at[k],
                    device_id=(my_x, my_y, peer),
                    device_id_type=pl.DeviceIdType.MESH,
                )
                ex.start()
                exchanges.append(ex)
                ex.wait_recv()
                out_ref[sl, :] = mid_ref[k] + buf_peer[k]
                fs = pltpu.make_async_remote_copy(
                    src_ref=out_ref.at[sl],
                    dst_ref=out_ref.at[sl],
                    send_sem=sem_final_send.at[k],
                    recv_sem=sem_final_recv.at[k],
                    device_id=(my_x, my_y, edge_nbr),
                    device_id_type=pl.DeviceIdType.MESH,
                )
                fs.start()
                final_sends.append(fs)
            for ex in exchanges:
                ex.wait_send()
            for fs in final_sends:
                fs.wait_send()

    return pl.pallas_call(
        body,
        out_shape=jax.ShapeDtypeStruct((m, d), jnp.float32),
        in_specs=[
            pl.BlockSpec(memory_space=pltpu.VMEM),
            pl.BlockSpec(memory_space=pltpu.VMEM),
        ],
        out_specs=pl.BlockSpec(memory_space=pltpu.VMEM),
        scratch_shapes=[
            pltpu.VMEM((m, d), jnp.float32),
            pltpu.VMEM((N_CHUNK, rows, d), jnp.float32),
            pltpu.VMEM((N_CHUNK, rows, d), jnp.float32),
            pltpu.VMEM((N_CHUNK, rows, d), jnp.float32),
            pltpu.SemaphoreType.DMA((N_CHUNK,)),
            pltpu.SemaphoreType.DMA((N_CHUNK,)),
            pltpu.SemaphoreType.DMA((N_CHUNK,)),
            pltpu.SemaphoreType.DMA((N_CHUNK,)),
            pltpu.SemaphoreType.DMA((N_CHUNK,)),
            pltpu.SemaphoreType.DMA((N_CHUNK,)),
        ],
        compiler_params=pltpu.CompilerParams(collective_id=0),
    )(dy, W)


# device time: 31242 ns/iter; 1.5178x vs baseline; 1.2969x over previous
import jax
import jax.numpy as jnp
from jax import lax
from jax.experimental import pallas as pl
from jax.experimental.pallas import tpu as pltpu

N_CHUNK = 4


def kernel(dy, W):
    m, _ = dy.shape
    d = W.shape[0]
    half = m // 2
    rows = half // N_CHUNK

    def body(dy_ref, w_ref, out_ref, acc_ref, mid_ref, buf_edge, buf_peer,
             sem_edge_send, sem_edge_recv, sem_peer_send, sem_peer_recv,
             sem_final_send, sem_final_recv, sem_x_send, sem_x_recv):
        my_x = lax.axis_index("x")
        my_y = lax.axis_index("y")
        my_z = lax.axis_index("z")
        is_edge = jnp.logical_or(my_z == 0, my_z == 3)
        base = my_x * half
        x_partner = 1 - my_x

        barrier = pltpu.get_barrier_semaphore()

        @pl.when(my_z < 3)
        def _():
            pl.semaphore_signal(
                barrier, inc=1,
                device_id=(my_x, my_y, my_z + 1),
                device_id_type=pl.DeviceIdType.MESH,
            )

        @pl.when(my_z > 0)
        def _():
            pl.semaphore_signal(
                barrier, inc=1,
                device_id=(my_x, my_y, my_z - 1),
                device_id_type=pl.DeviceIdType.MESH,
            )

        pl.semaphore_signal(
            barrier, inc=1,
            device_id=(x_partner, my_y, my_z),
            device_id_type=pl.DeviceIdType.MESH,
        )
        pl.semaphore_wait(barrier, jnp.where(is_edge, 2, 3))

        acc_ref[...] = lax.dot_general(
            dy_ref[pl.ds(base, half), :], w_ref[...],
            dimension_numbers=(((1,), (1,)), ((), ())),
            preferred_element_type=jnp.float32,
        )

        def x_forward(k):
            sl = pl.ds(base + k * rows, rows)
            fx = pltpu.make_async_remote_copy(
                src_ref=out_ref.at[sl],
                dst_ref=out_ref.at[sl],
                send_sem=sem_x_send.at[k],
                recv_sem=sem_x_recv.at[k],
                device_id=(x_partner, my_y, my_z),
                device_id_type=pl.DeviceIdType.MESH,
            )
            fx.start()
            return fx

        @pl.when(is_edge)
        def _():
            inner = jnp.where(my_z == 0, 1, 2)
            sends = []
            for k in range(N_CHUNK):
                s = pltpu.make_async_remote_copy(
                    src_ref=acc_ref.at[pl.ds(k * rows, rows)],
                    dst_ref=buf_edge.at[k],
                    send_sem=sem_edge_send.at[k],
                    recv_sem=sem_edge_recv.at[k],
                    device_id=(my_x, my_y, inner),
                    device_id_type=pl.DeviceIdType.MESH,
                )
                s.start()
                sends.append(s)
            xchs = []
            for k in range(N_CHUNK):
                sl = pl.ds(base + k * rows, rows)
                r = pltpu.make_async_remote_copy(
                    src_ref=acc_ref.at[pl.ds(k * rows, rows)],
                    dst_ref=out_ref.at[sl],
                    send_sem=sem_final_send.at[k],
                    recv_sem=sem_final_recv.at[k],
                    device_id=(my_x, my_y, inner),
                    device_id_type=pl.DeviceIdType.MESH,
                )
                r.wait_recv()
                xchs.append(x_forward(k))
            for s in sends:
                s.wait_send()
            for f in xchs:
                f.wait()

        @pl.when(jnp.logical_not(is_edge))
        def _():
            edge_nbr = jnp.where(my_z == 1, 0, 3)
            peer = jnp.where(my_z == 1, 2, 1)
            sends = []
            xchs = []
            for k in range(N_CHUNK):
                lsl = pl.ds(k * rows, rows)
                osl = pl.ds(base + k * rows, rows)
                re = pltpu.make_async_remote_copy(
                    src_ref=acc_ref.at[lsl],
                    dst_ref=buf_edge.at[k],
                    send_sem=sem_edge_send.at[k],
                    recv_sem=sem_edge_recv.at[k],
                    device_id=(my_x, my_y, edge_nbr),
                    device_id_type=pl.DeviceIdType.MESH,
                )
                re.wait_recv()
                mid_ref[k] = acc_ref[lsl, :] + buf_edge[k]
                ex = pltpu.make_async_remote_copy(
                    src_ref=mid_ref.at[k],
                    dst_ref=buf_peer.at[k],
                    send_sem=sem_peer_send.at[k],
                    recv_sem=sem_peer_recv.at[k],
                    device_id=(my_x, my_y, peer),
                    device_id_type=pl.DeviceIdType.MESH,
                )
                ex.start()
                sends.append(ex)
                ex.wait_recv()
                out_ref[osl, :] = mid_ref[k] + buf_peer[k]
                fs = pltpu.make_async_remote_copy(
                    src_ref=out_ref.at[osl],
                    dst_ref=out_ref.at[osl],
                    send_sem=sem_final_send.at[k],
                    recv_sem=sem_final_recv.at[k],
                    device_id=(my_x, my_y, edge_nbr),
                    device_id_type=pl.DeviceIdType.MESH,
                )
                fs.start()
                sends.append(fs)
                xchs.append(x_forward(k))
            for s in sends:
                s.wait_send()
            for f in xchs:
                f.wait()

    return pl.pallas_call(
        body,
        out_shape=jax.ShapeDtypeStruct((m, d), jnp.float32),
        in_specs=[
            pl.BlockSpec(memory_space=pltpu.VMEM),
            pl.BlockSpec(memory_space=pltpu.VMEM),
        ],
        out_specs=pl.BlockSpec(memory_space=pltpu.VMEM),
        scratch_shapes=[
            pltpu.VMEM((half, d), jnp.float32),
            pltpu.VMEM((N_CHUNK, rows, d), jnp.float32),
            pltpu.VMEM((N_CHUNK, rows, d), jnp.float32),
            pltpu.VMEM((N_CHUNK, rows, d), jnp.float32),
            pltpu.SemaphoreType.DMA((N_CHUNK,)),
            pltpu.SemaphoreType.DMA((N_CHUNK,)),
            pltpu.SemaphoreType.DMA((N_CHUNK,)),
            pltpu.SemaphoreType.DMA((N_CHUNK,)),
            pltpu.SemaphoreType.DMA((N_CHUNK,)),
            pltpu.SemaphoreType.DMA((N_CHUNK,)),
            pltpu.SemaphoreType.DMA((N_CHUNK,)),
            pltpu.SemaphoreType.DMA((N_CHUNK,)),
        ],
        compiler_params=pltpu.CompilerParams(collective_id=0),
    )(dy, W)
